# baseline (speedup 1.0000x reference)
import functools

import jax
import jax.numpy as jnp
from jax import lax
from jax.experimental import pallas as pl
from jax.experimental.pallas import tpu as pltpu
from jax.experimental.pallas import tpu_sc as plsc

_D = 128
_R = 256
_NBUF = 2
_NW = 32


def _sc_body(x_hbm, p_hbm, o_hbm, xbuf, obuf, pbuf, tbuf, insem, outsem,
             *, rows_per_w, nchunk):
    wid = lax.axis_index("s") * 2 + lax.axis_index("c")
    row0 = wid * rows_per_w

    pltpu.sync_copy(p_hbm, pbuf)
    p_vecs = [pbuf[pl.ds(16 * j, 16)] for j in range(8)]
    iota16 = lax.iota(jnp.int32, 16)
    col_base = iota16 * 16

    def in_copy(chunk, slot):
        return pltpu.make_async_copy(
            x_hbm.at[pl.ds(row0 + chunk * _R, _R)], xbuf.at[slot],
            insem.at[slot])

    for s in range(_NBUF):
        in_copy(s, s).start()

    def compute_chunk(chunk, slot):
        def group(g, _):
            for l in range(16):
                row = g * 16 + l
                prods = [xbuf[slot, row, pl.ds(16 * j, 16)] * p_vecs[j]
                         for j in range(8)]
                t = ((prods[0] + prods[1]) + (prods[2] + prods[3])) + \
                    ((prods[4] + prods[5]) + (prods[6] + prods[7]))
                tbuf[pl.ds(l * 16, 16)] = t
            acc = plsc.load_gather(tbuf, [col_base])
            for c in range(1, 16):
                acc = acc + plsc.load_gather(tbuf, [col_base + c])
            obuf[pl.ds(chunk * _R + g * 16, 16)] = acc
            return 0

        lax.fori_loop(0, _R // 16, group, 0)

    def outer(i, _):
        for b in range(_NBUF):
            chunk = i * _NBUF + b
            in_copy(chunk, b).wait()
            compute_chunk(chunk, b)

            @pl.when(chunk + _NBUF < nchunk)
            def _():
                in_copy(chunk + _NBUF, b).start()
        return 0

    lax.fori_loop(0, nchunk // _NBUF, outer, 0)
    final = pltpu.make_async_copy(
        obuf, o_hbm.at[pl.ds(row0, rows_per_w)], outsem)
    final.start()
    final.wait()


def sc_matvec(xf, p):
    rows = xf.shape[0]
    rows_per_w = rows // _NW
    nchunk = rows_per_w // _R
    mesh = plsc.VectorSubcoreMesh(core_axis_name="c", subcore_axis_name="s")
    f = pl.kernel(
        functools.partial(_sc_body, rows_per_w=rows_per_w, nchunk=nchunk),
        out_type=jax.ShapeDtypeStruct((rows,), jnp.float32),
        mesh=mesh,
        compiler_params=pltpu.CompilerParams(needs_layout_passes=False),
        scratch_types=[
            pltpu.VMEM((_NBUF, _R, _D), jnp.float32),
            pltpu.VMEM((rows_per_w,), jnp.float32),
            pltpu.VMEM((_D,), jnp.float32),
            pltpu.VMEM((256,), jnp.float32),
            pltpu.SemaphoreType.DMA((_NBUF,)),
            pltpu.SemaphoreType.DMA,
        ],
    )
    return f(xf, p)


def kernel(x, projection):
    b, n, d = x.shape
    xf = x.reshape(b * n, d)
    out = sc_matvec(xf, projection.reshape(d))
    return out.reshape(b, n, 1)


# speedup vs baseline: 2.5742x; 2.5742x over previous
import functools

import jax
import jax.numpy as jnp
from jax import lax
from jax.experimental import pallas as pl
from jax.experimental.pallas import tpu as pltpu
from jax.experimental.pallas import tpu_sc as plsc

_D = 128

_TOTAL_GROUPS = 4096
_SC_GROUPS = 768
_TC_GROUPS = _TOTAL_GROUPS - _SC_GROUPS

_G = 64
_NBUF = 4

_R = 256
_SC_NBUF = 2
_NW = 32


def _tc_body(x_hbm, p_ref, o_hbm, xbuf, obuf, insem, outsem, *,
             base_group, nstep):
    def in_copy(step, slot):
        return pltpu.make_async_copy(
            x_hbm.at[pl.ds(base_group + step * _G, _G)], xbuf.at[slot],
            insem.at[slot])

    for s in range(_NBUF):
        in_copy(s, s).start()

    def outer(i, _):
        for b in range(_NBUF):
            step = i * _NBUF + b
            in_copy(step, b).wait()
            y = lax.dot_general(
                p_ref[...], xbuf[b],
                dimension_numbers=(((2,), (2,)), ((0,), (0,))),
                preferred_element_type=jnp.float32,
            )
            obuf[pl.ds(step * _G, _G)] = y.reshape(_G, _D)

            @pl.when(step + _NBUF < nstep)
            def _():
                in_copy(step + _NBUF, b).start()
        return 0

    lax.fori_loop(0, nstep // _NBUF, outer, 0)
    final = pltpu.make_async_copy(obuf, o_hbm, outsem)
    final.start()
    final.wait()


def _tc_matvec(xg, pb):
    nstep = _TC_GROUPS // _G
    out = pl.pallas_call(
        functools.partial(_tc_body, base_group=_SC_GROUPS, nstep=nstep),
        in_specs=[
            pl.BlockSpec(memory_space=pl.ANY),
            pl.BlockSpec(memory_space=pltpu.VMEM),
        ],
        out_specs=pl.BlockSpec(memory_space=pl.ANY),
        out_shape=jax.ShapeDtypeStruct((_TC_GROUPS, _D), jnp.float32),
        scratch_shapes=[
            pltpu.VMEM((_NBUF, _G, _D, _D), jnp.float32),
            pltpu.VMEM((_TC_GROUPS, _D), jnp.float32),
            pltpu.SemaphoreType.DMA((_NBUF,)),
            pltpu.SemaphoreType.DMA,
        ],
    )(xg, pb)
    return out


def _sc_body(x_hbm, p_hbm, o_hbm, xbuf, obuf, pbuf, tbuf, insem, outsem,
             *, rows_per_w, nchunk):
    wid = lax.axis_index("s") * 2 + lax.axis_index("c")
    row0 = wid * rows_per_w

    pltpu.sync_copy(p_hbm, pbuf)
    p_vecs = [pbuf[pl.ds(16 * j, 16)] for j in range(8)]
    iota16 = lax.iota(jnp.int32, 16)
    col_base = iota16 * 16

    def in_copy(chunk, slot):
        return pltpu.make_async_copy(
            x_hbm.at[pl.ds(row0 + chunk * _R, _R)], xbuf.at[slot],
            insem.at[slot])

    for s in range(_SC_NBUF):
        in_copy(s, s).start()

    def compute_chunk(chunk, slot):
        def group(g, _):
            for l in range(16):
                row = g * 16 + l
                prods = [xbuf[slot, row, pl.ds(16 * j, 16)] * p_vecs[j]
                         for j in range(8)]
                t = ((prods[0] + prods[1]) + (prods[2] + prods[3])) + \
                    ((prods[4] + prods[5]) + (prods[6] + prods[7]))
                tbuf[pl.ds(l * 16, 16)] = t
            acc = plsc.load_gather(tbuf, [col_base])
            for c in range(1, 16):
                acc = acc + plsc.load_gather(tbuf, [col_base + c])
            obuf[pl.ds(chunk * _R + g * 16, 16)] = acc
            return 0

        lax.fori_loop(0, _R // 16, group, 0)

    def outer(i, _):
        for b in range(_SC_NBUF):
            chunk = i * _SC_NBUF + b
            in_copy(chunk, b).wait()
            compute_chunk(chunk, b)

            @pl.when(chunk + _SC_NBUF < nchunk)
            def _():
                in_copy(chunk + _SC_NBUF, b).start()
        return 0

    lax.fori_loop(0, nchunk // _SC_NBUF, outer, 0)
    final = pltpu.make_async_copy(
        obuf, o_hbm.at[pl.ds(row0, rows_per_w)], outsem)
    final.start()
    final.wait()


def _sc_matvec(xf, p):
    rows = _SC_GROUPS * _D
    rows_per_w = rows // _NW
    nchunk = rows_per_w // _R
    mesh = plsc.VectorSubcoreMesh(core_axis_name="c", subcore_axis_name="s")
    f = pl.kernel(
        functools.partial(_sc_body, rows_per_w=rows_per_w, nchunk=nchunk),
        out_type=jax.ShapeDtypeStruct((rows,), jnp.float32),
        mesh=mesh,
        compiler_params=pltpu.CompilerParams(needs_layout_passes=False),
        scratch_types=[
            pltpu.VMEM((_SC_NBUF, _R, _D), jnp.float32),
            pltpu.VMEM((rows_per_w,), jnp.float32),
            pltpu.VMEM((_D,), jnp.float32),
            pltpu.VMEM((256,), jnp.float32),
            pltpu.SemaphoreType.DMA((_SC_NBUF,)),
            pltpu.SemaphoreType.DMA,
        ],
    )
    return f(xf, p)


def kernel(x, projection):
    b, n, d = x.shape
    rows = b * n
    xf = x.reshape(rows, d)
    xg = x.reshape(rows // d, d, d)
    pb = jnp.broadcast_to(projection.reshape(1, 1, d), (_G, 1, d))
    sc_out = _sc_matvec(xf, projection.reshape(d))
    tc_out = _tc_matvec(xg, pb)
    out = jnp.concatenate([sc_out, tc_out.reshape(-1)])
    return out.reshape(b, n, 1)


# speedup vs baseline: 2.6052x; 1.0120x over previous
import functools

import jax
import jax.numpy as jnp
from jax import lax
from jax.experimental import pallas as pl
from jax.experimental.pallas import tpu as pltpu
from jax.experimental.pallas import tpu_sc as plsc

_D = 128

_TOTAL_GROUPS = 4096
_SC_GROUPS = 256
_TC_GROUPS = _TOTAL_GROUPS - _SC_GROUPS

_G = 64
_NBUF = 4

_R = 256
_SC_NBUF = 2
_NW = 32


def _tc_body(x_hbm, p_ref, o_hbm, xbuf, obuf, insem, outsem, *,
             base_group, nstep):
    def in_copy(step, slot):
        return pltpu.make_async_copy(
            x_hbm.at[pl.ds(base_group + step * _G, _G)], xbuf.at[slot],
            insem.at[slot])

    for s in range(_NBUF):
        in_copy(s, s).start()

    def outer(i, _):
        for b in range(_NBUF):
            step = i * _NBUF + b
            in_copy(step, b).wait()
            y = lax.dot_general(
                p_ref[...], xbuf[b],
                dimension_numbers=(((2,), (2,)), ((0,), (0,))),
                preferred_element_type=jnp.float32,
            )
            obuf[pl.ds(step * _G, _G)] = y.reshape(_G, _D)

            @pl.when(step + _NBUF < nstep)
            def _():
                in_copy(step + _NBUF, b).start()
        return 0

    lax.fori_loop(0, nstep // _NBUF, outer, 0)
    final = pltpu.make_async_copy(obuf, o_hbm, outsem)
    final.start()
    final.wait()


def _tc_matvec(xg, pb):
    nstep = _TC_GROUPS // _G
    out = pl.pallas_call(
        functools.partial(_tc_body, base_group=_SC_GROUPS, nstep=nstep),
        in_specs=[
            pl.BlockSpec(memory_space=pl.ANY),
            pl.BlockSpec(memory_space=pltpu.VMEM),
        ],
        out_specs=pl.BlockSpec(memory_space=pl.ANY),
        out_shape=jax.ShapeDtypeStruct((_TC_GROUPS, _D), jnp.float32),
        scratch_shapes=[
            pltpu.VMEM((_NBUF, _G, _D, _D), jnp.float32),
            pltpu.VMEM((_TC_GROUPS, _D), jnp.float32),
            pltpu.SemaphoreType.DMA((_NBUF,)),
            pltpu.SemaphoreType.DMA,
        ],
    )(xg, pb)
    return out


def _sc_body(x_hbm, p_hbm, o_hbm, xbuf, obuf, pbuf, tbuf, insem, outsem,
             *, rows_per_w, nchunk):
    wid = lax.axis_index("s") * 2 + lax.axis_index("c")
    row0 = wid * rows_per_w

    pltpu.sync_copy(p_hbm, pbuf)
    p_vecs = [pbuf[pl.ds(16 * j, 16)] for j in range(8)]
    iota16 = lax.iota(jnp.int32, 16)
    col_base = iota16 * 16

    def in_copy(chunk, slot):
        return pltpu.make_async_copy(
            x_hbm.at[pl.ds(row0 + chunk * _R, _R)], xbuf.at[slot],
            insem.at[slot])

    for s in range(_SC_NBUF):
        in_copy(s, s).start()

    def compute_chunk(chunk, slot):
        def group(g, _):
            for l in range(16):
                row = g * 16 + l
                prods = [xbuf[slot, row, pl.ds(16 * j, 16)] * p_vecs[j]
                         for j in range(8)]
                t = ((prods[0] + prods[1]) + (prods[2] + prods[3])) + \
                    ((prods[4] + prods[5]) + (prods[6] + prods[7]))
                tbuf[pl.ds(l * 16, 16)] = t
            acc = plsc.load_gather(tbuf, [col_base])
            for c in range(1, 16):
                acc = acc + plsc.load_gather(tbuf, [col_base + c])
            obuf[pl.ds(chunk * _R + g * 16, 16)] = acc
            return 0

        lax.fori_loop(0, _R // 16, group, 0)

    def outer(i, _):
        for b in range(_SC_NBUF):
            chunk = i * _SC_NBUF + b
            in_copy(chunk, b).wait()
            compute_chunk(chunk, b)

            @pl.when(chunk + _SC_NBUF < nchunk)
            def _():
                in_copy(chunk + _SC_NBUF, b).start()
        return 0

    lax.fori_loop(0, nchunk // _SC_NBUF, outer, 0)
    final = pltpu.make_async_copy(
        obuf, o_hbm.at[pl.ds(row0, rows_per_w)], outsem)
    final.start()
    final.wait()


def _sc_matvec(xf, p):
    rows = _SC_GROUPS * _D
    rows_per_w = rows // _NW
    nchunk = rows_per_w // _R
    mesh = plsc.VectorSubcoreMesh(core_axis_name="c", subcore_axis_name="s")
    f = pl.kernel(
        functools.partial(_sc_body, rows_per_w=rows_per_w, nchunk=nchunk),
        out_type=jax.ShapeDtypeStruct((rows,), jnp.float32),
        mesh=mesh,
        compiler_params=pltpu.CompilerParams(needs_layout_passes=False),
        scratch_types=[
            pltpu.VMEM((_SC_NBUF, _R, _D), jnp.float32),
            pltpu.VMEM((rows_per_w,), jnp.float32),
            pltpu.VMEM((_D,), jnp.float32),
            pltpu.VMEM((256,), jnp.float32),
            pltpu.SemaphoreType.DMA((_SC_NBUF,)),
            pltpu.SemaphoreType.DMA,
        ],
    )
    return f(xf, p)


def kernel(x, projection):
    b, n, d = x.shape
    rows = b * n
    xf = x.reshape(rows, d)
    xg = x.reshape(rows // d, d, d)
    pb = jnp.broadcast_to(projection.reshape(1, 1, d), (_G, 1, d))
    sc_out = _sc_matvec(xf, projection.reshape(d))
    tc_out = _tc_matvec(xg, pb)
    out = jnp.concatenate([sc_out, tc_out.reshape(-1)])
    return out.reshape(b, n, 1)


# speedup vs baseline: 2.9802x; 1.1440x over previous
import functools

import jax
import jax.numpy as jnp
from jax import lax
from jax.experimental import pallas as pl
from jax.experimental.pallas import tpu as pltpu

_G = 64
_NBUF = 8
_D = 128


def _body(x_hbm, p_ref, o_hbm, xbuf, obuf, insem, outsem, *, nstep):
    def in_copy(step, slot):
        return pltpu.make_async_copy(
            x_hbm.at[pl.ds(step * _G, _G)], xbuf.at[slot], insem.at[slot])

    for s in range(_NBUF):
        in_copy(s, s).start()

    def outer(i, _):
        for b in range(_NBUF):
            step = i * _NBUF + b
            in_copy(step, b).wait()
            y = lax.dot_general(
                p_ref[...], xbuf[b],
                dimension_numbers=(((2,), (2,)), ((0,), (0,))),
                preferred_element_type=jnp.float32,
            )
            obuf[pl.ds(step * _G, _G)] = y.reshape(_G, _D)

            @pl.when(step + _NBUF < nstep)
            def _():
                in_copy(step + _NBUF, b).start()
        return 0

    lax.fori_loop(0, nstep // _NBUF, outer, 0)
    final = pltpu.make_async_copy(obuf, o_hbm, outsem)
    final.start()
    final.wait()


def kernel(x, projection):
    b, n, d = x.shape
    rows = b * n
    groups = rows // d
    nstep = groups // _G
    xf = x.reshape(groups, d, d)
    pb = jnp.broadcast_to(projection.reshape(1, 1, d), (_G, 1, d))
    out = pl.pallas_call(
        functools.partial(_body, nstep=nstep),
        in_specs=[
            pl.BlockSpec(memory_space=pl.ANY),
            pl.BlockSpec(memory_space=pltpu.VMEM),
        ],
        out_specs=pl.BlockSpec(memory_space=pl.ANY),
        out_shape=jax.ShapeDtypeStruct((groups, d), jnp.float32),
        scratch_shapes=[
            pltpu.VMEM((_NBUF, _G, d, d), jnp.float32),
            pltpu.VMEM((groups, d), jnp.float32),
            pltpu.SemaphoreType.DMA((_NBUF,)),
            pltpu.SemaphoreType.DMA,
        ],
    )(xf, pb)
    return out.reshape(b, n, 1)


# speedup vs baseline: 2.9876x; 1.0025x over previous
import functools

import jax
import jax.numpy as jnp
from jax import lax
from jax.experimental import pallas as pl
from jax.experimental.pallas import tpu as pltpu

_G = 32
_NBUF = 16
_D = 128


def _body(x_hbm, p_ref, o_hbm, xbuf, obuf, insem, outsem, *, nstep):
    def in_copy(step, slot):
        return pltpu.make_async_copy(
            x_hbm.at[pl.ds(step * _G, _G)], xbuf.at[slot], insem.at[slot])

    for s in range(_NBUF):
        in_copy(s, s).start()

    def outer(i, _):
        for b in range(_NBUF):
            step = i * _NBUF + b
            in_copy(step, b).wait()
            y = lax.dot_general(
                p_ref[...], xbuf[b],
                dimension_numbers=(((2,), (2,)), ((0,), (0,))),
                preferred_element_type=jnp.float32,
            )
            obuf[pl.ds(step * _G, _G)] = y.reshape(_G, _D)

            @pl.when(step + _NBUF < nstep)
            def _():
                in_copy(step + _NBUF, b).start()
        return 0

    lax.fori_loop(0, nstep // _NBUF, outer, 0)
    final = pltpu.make_async_copy(obuf, o_hbm, outsem)
    final.start()
    final.wait()


def kernel(x, projection):
    b, n, d = x.shape
    rows = b * n
    groups = rows // d
    nstep = groups // _G
    xf = x.reshape(groups, d, d)
    pb = jnp.broadcast_to(projection.reshape(1, 1, d), (_G, 1, d))
    out = pl.pallas_call(
        functools.partial(_body, nstep=nstep),
        in_specs=[
            pl.BlockSpec(memory_space=pl.ANY),
            pl.BlockSpec(memory_space=pltpu.VMEM),
        ],
        out_specs=pl.BlockSpec(memory_space=pl.ANY),
        out_shape=jax.ShapeDtypeStruct((groups, d), jnp.float32),
        scratch_shapes=[
            pltpu.VMEM((_NBUF, _G, d, d), jnp.float32),
            pltpu.VMEM((groups, d), jnp.float32),
            pltpu.SemaphoreType.DMA((_NBUF,)),
            pltpu.SemaphoreType.DMA,
        ],
    )(xf, pb)
    return out.reshape(b, n, 1)
